# baseline (speedup 1.0000x reference)
import functools

import jax
import jax.numpy as jnp
from jax import lax
from jax.experimental import pallas as pl
from jax.experimental.pallas import tpu as pltpu
from jax.experimental.pallas import tpu_sc as plsc

MAXD = 10
V = MAXD + 1
D = 64
N_SIDE = 1024
B = N_SIDE * N_SIDE
NC = 2
NS = 16
NW = NC * NS
K = 128
NKC = B // (NW * K)
L = 16
NBUF = 4


def _body(raw_hbm, table_hbm, out_hbm, idx_v, table_sh, *bufs_and_sems):
    rows = bufs_and_sems[:NBUF]
    sg = bufs_and_sems[NBUF : 2 * NBUF]
    so = bufs_and_sems[2 * NBUF : 3 * NBUF]

    sid = lax.axis_index("s")
    wid = sid * NC + lax.axis_index("c")
    row0 = wid * NKC

    @pl.when(sid == 0)
    def _():
        pltpu.sync_copy(table_hbm, table_sh)

    pltpu.sync_copy(raw_hbm.at[pl.ds(row0, NKC)], idx_v)

    def clamp_row(j, carry):
        for t in range(K // L):
            sl = pl.ds(t * L, L)
            v = idx_v[j, sl]
            idx_v[j, sl] = jnp.minimum(jnp.maximum(v, 0), MAXD)
        return carry

    lax.fori_loop(0, NKC, clamp_row, 0)

    plsc.subcore_barrier()

    def fire_gather(j, b):
        pltpu.async_copy(table_sh.at[idx_v.at[j]], rows[b], sg[b])

    def fire_write(j, b):
        pltpu.async_copy(rows[b], out_hbm.at[pl.ds((row0 + j) * K, K)], so[b])

    def wait_gather(j, b):
        pltpu.make_async_copy(table_sh.at[idx_v.at[j]], rows[b], sg[b]).wait()

    def wait_write(j, b):
        pltpu.make_async_copy(
            rows[b], out_hbm.at[pl.ds((row0 + j) * K, K)], so[b]
        ).wait()

    for b in range(NBUF):
        fire_gather(b, b)

    def outer(gi, carry):
        g = gi * NBUF
        for b in range(NBUF):
            j = g + b
            wait_gather(j, b)
            fire_write(j, b)
            wait_write(j, b)

            @pl.when(j + NBUF < NKC)
            def _():
                fire_gather(j + NBUF, b)

        return carry

    lax.fori_loop(0, NKC // NBUF, outer, 0)


_gather_call = functools.partial(
    pl.kernel,
    out_type=jax.ShapeDtypeStruct((B, D), jnp.float32),
    mesh=plsc.VectorSubcoreMesh(
        core_axis_name="c", subcore_axis_name="s", num_cores=NC, num_subcores=NS
    ),
    compiler_params=pltpu.CompilerParams(use_tc_tiling_on_sc=False),
    scratch_types=(
        [
            pltpu.VMEM((NKC, K), jnp.int32),
            pltpu.VMEM_SHARED((V, D), jnp.float32),
        ]
        + [pltpu.VMEM((K, D), jnp.float32)] * NBUF
        + [pltpu.SemaphoreType.DMA] * (2 * NBUF)
    ),
)(_body)


def kernel(distance_matrix, table):
    raw = distance_matrix.reshape(B // K, K)
    out = _gather_call(raw, table)
    return out.reshape(N_SIDE, N_SIDE, D)
